# baseline (speedup 1.0000x reference)
import functools

import jax
import jax.numpy as jnp
from jax import lax
from jax.experimental import pallas as pl
from jax.experimental.pallas import tpu as pltpu
from jax.experimental.pallas import tpu_sc as plsc

BATCH = 4096
FEAT = 128
ROW_TILE = 512


def _make_sc_gather():
    info = plsc.get_sparse_core_info()
    nc, ns = info.num_cores, info.num_subcores
    nw = nc * ns
    b_per_w = BATCH // nw
    mesh = plsc.VectorSubcoreMesh(core_axis_name="c", subcore_axis_name="s")

    @functools.partial(
        pl.kernel,
        mesh=mesh,
        out_type=jax.ShapeDtypeStruct((BATCH, FEAT), jnp.float32),
        scratch_types=[
            pltpu.VMEM((b_per_w,), jnp.int32),
            pltpu.VMEM((b_per_w, FEAT), jnp.float32),
            pltpu.SemaphoreType.DMA,
        ],
    )
    def gather(centers_hbm, labels_hbm, out_hbm, idx_v, rows_v, sem):
        wid = lax.axis_index("s") * nc + lax.axis_index("c")
        base = wid * b_per_w
        pltpu.sync_copy(labels_hbm.at[pl.ds(base, b_per_w)], idx_v)
        pltpu.async_copy(centers_hbm.at[idx_v], rows_v, sem).wait()
        pltpu.sync_copy(rows_v, out_hbm.at[pl.ds(base, b_per_w)])

    return gather


_sc_gather = _make_sc_gather()


def _loss_body(x_ref, bc_ref, lab_row_ref, lab_all_ref, out_ref):
    i = pl.program_id(0)
    x = x_ref[...]
    bc = bc_ref[...]
    sim = lax.dot_general(
        x, bc, (((1,), (1,)), ((), ())),
        preferred_element_type=jnp.float32,
    )
    sim = jnp.maximum(sim, 0.0)
    li = lab_row_ref[0, :]
    lj = lab_all_ref[0, :]
    tgt = (li[:, None] == lj[None, :]).astype(jnp.float32)
    d = sim - tgt
    part = jnp.sum(d * d)

    @pl.when(i == 0)
    def _():
        out_ref[0, 0] = 0.0

    out_ref[0, 0] += part


def _fused_loss(x, batch_centers, labels2d):
    grid = (BATCH // ROW_TILE,)
    return pl.pallas_call(
        _loss_body,
        grid=grid,
        in_specs=[
            pl.BlockSpec((ROW_TILE, FEAT), lambda i: (i, 0)),
            pl.BlockSpec((BATCH, FEAT), lambda i: (0, 0)),
            pl.BlockSpec((1, ROW_TILE), lambda i: (0, i)),
            pl.BlockSpec((1, BATCH), lambda i: (0, 0)),
        ],
        out_specs=pl.BlockSpec(
            (1, 1), lambda i: (0, 0), memory_space=pltpu.SMEM
        ),
        out_shape=jax.ShapeDtypeStruct((1, 1), jnp.float32),
    )(x, batch_centers, labels2d, labels2d)


@jax.jit
def kernel(x, labels, centers):
    batch_centers = _sc_gather(centers, labels)
    labels2d = labels.reshape(1, BATCH)
    out = _fused_loss(x, batch_centers, labels2d)
    return out[0, 0]
